# baseline (speedup 1.0000x reference)
import functools

import jax
import jax.numpy as jnp
from jax import lax
from jax.experimental import pallas as pl
from jax.experimental.pallas import tpu as pltpu
from jax.experimental.pallas import tpu_sc as plsc

_L = 16
_CH = 128


@functools.cache
def _build(NJ, NI, V2, D):
    info = plsc.get_sparse_core_info()
    NC, NS = info.num_cores, info.num_subcores
    NW = NC * NS
    NTI = NI // _CH
    assert NTI == NW and NJ >= 4 and D == 64
    mesh = plsc.VectorSubcoreMesh(core_axis_name="c", subcore_axis_name="s")

    @functools.partial(
        pl.kernel,
        out_type=jax.ShapeDtypeStruct((NJ, D // 8, NTI, 8, _CH), jnp.float32),
        mesh=mesh,
        scratch_types=[
            pltpu.VMEM((NJ, _CH), jnp.int32),
            pltpu.VMEM((2, _CH), jnp.int32),
            pltpu.VMEM((2, _CH, 2 * D), jnp.float32),
            pltpu.VMEM((2, D // 8, 8, _CH), jnp.float32),
            pltpu.SemaphoreType.DMA,
            pltpu.SemaphoreType.DMA,
        ],
        compiler_params=pltpu.CompilerParams(
            use_tc_tiling_on_sc=False, needs_layout_passes=False),
    )
    def gather_kernel(xt_hbm, tab_hbm, out_hbm, idx_v, rid_v, gbuf, tbuf,
                      gsem, ssem):
        wid = lax.axis_index("s") * NC + lax.axis_index("c")
        i0 = wid * _CH
        pltpu.sync_copy(xt_hbm.at[:, pl.ds(i0, _CH)], idx_v)
        lanes = lax.iota(jnp.int32, _L)

        def fill_rids_and_gather(j, b):
            for t0 in range(0, _CH, _L):
                r = idx_v[j, pl.ds(t0, _L)]
                rid_v[b, pl.ds(t0, _L)] = jax.lax.shift_right_logical(r, 1)
            pltpu.async_copy(tab_hbm.at[rid_v.at[b]], gbuf.at[b], gsem)

        for b in range(2):
            fill_rids_and_gather(b, b)

        def chunk(j, carry):
            b = j & 1
            gb = gbuf.at[b]
            tb = tbuf.at[b]
            pltpu.make_async_copy(
                tab_hbm.at[pl.ds(0, _CH)], gb, gsem).wait()

            @pl.when(j >= 2)
            def _():
                pltpu.make_async_copy(tb, out_hbm.at[0, :, 0], ssem).wait()

            for t0 in range(0, _CH, _L):
                r = idx_v[j, pl.ds(t0, _L)]
                rows = lanes + t0
                cb = (r & 1) * D
                for c in range(D):
                    v = plsc.load_gather(gb, [rows, cb + c])
                    tb[c // 8, c % 8, pl.ds(t0, _L)] = v

            pltpu.async_copy(tb, out_hbm.at[j, :, wid], ssem)

            @pl.when(j + 2 < NJ)
            def _():
                fill_rids_and_gather(j + 2, b)

            return carry

        lax.fori_loop(0, NJ, chunk, 0)
        for b in range(2):
            pltpu.make_async_copy(
                tbuf.at[b], out_hbm.at[0, :, 0], ssem).wait()

    return gather_kernel


def kernel(x, table):
    S0, S1 = x.shape
    V, D = table.shape
    xt = x.T.astype(jnp.int32)
    tabw = table.reshape(V // 2, 2 * D)
    outp = _build(S1, S0, V // 2, D)(xt, tabw)
    return outp.transpose(2, 4, 0, 1, 3).reshape(S0, S1, D)
